# baseline (speedup 1.0000x reference)
import functools

import jax
import jax.numpy as jnp
from jax import lax
from jax.experimental import pallas as pl
from jax.experimental.pallas import tpu as pltpu
from jax.experimental.pallas import tpu_sc as plsc

_LANES = 16

_GATHER_DNUMS = lax.GatherDimensionNumbers(
    offset_dims=(), collapsed_slice_dims=(0,), start_index_map=(0,))


def _shuffle(x, idx):
    return lax.gather(x, idx[:, None], _GATHER_DNUMS, (1,),
                      indices_are_sorted=False, unique_indices=True,
                      mode=lax.GatherScatterMode.PROMISE_IN_BOUNDS)


@functools.lru_cache(maxsize=None)
def _build_sc_kernel(B, S, V):
    info = plsc.get_sparse_core_info()
    num_cores = info.num_cores
    num_workers = info.num_cores * info.num_subcores
    assert B == num_workers, (B, num_workers)
    assert V % _LANES == 0
    mesh = plsc.VectorSubcoreMesh(core_axis_name="c", subcore_axis_name="s")

    @functools.partial(
        pl.kernel,
        mesh=mesh,
        out_type=jax.ShapeDtypeStruct((B, _LANES), jnp.int32),
        scratch_types=(
            pltpu.VMEM((V,), jnp.float32),
            pltpu.VMEM((_LANES,), jnp.int32),
            pltpu.SemaphoreType.DMA,
        ),
    )
    def sc_kernel(probas_hbm, cand_hbm, buf, idx_buf, in_sem):
        wid = lax.axis_index("s") * num_cores + lax.axis_index("c")

        pltpu.async_copy(probas_hbm.at[wid, S - 1], buf, in_sem).wait()

        lane = lax.iota(jnp.int32, _LANES)
        intmax = jnp.int32(2**31 - 1)

        K = 5
        group = K * _LANES
        assert V % group == 0

        def body(j, carry):
            ms, mis, iv = carry
            new_ms, new_mis = [], []
            for k in range(K):
                v = buf[pl.ds(j * group + k * _LANES, _LANES)]
                pred = v > ms[k]
                new_ms.append(jnp.where(pred, v, ms[k]))
                new_mis.append(jnp.where(pred, iv, mis[k]))
            return tuple(new_ms), tuple(new_mis), iv + group

        ms0 = tuple(jnp.full((_LANES,), -jnp.inf, jnp.float32) for _ in range(K))
        mis0 = tuple(jnp.zeros((_LANES,), jnp.int32) for _ in range(K))
        ms, mis, _ = lax.fori_loop(0, V // group, body,
                                   (ms0, mis0, jnp.zeros((_LANES,), jnp.int32)),
                                   unroll=4)

        fis = [mis[k] + (k * _LANES) + lane for k in range(K)]

        mlane = ms[0]
        for k in range(1, K):
            mlane = jnp.maximum(mlane, ms[k])
        cand = intmax
        for k in range(K):
            cand = jnp.minimum(cand, jnp.where(ms[k] == mlane, fis[k], intmax))

        gm = mlane
        for k in (1, 2, 4, 8):
            gm = jnp.maximum(gm, _shuffle(gm, lane ^ k))
        cand = jnp.where(mlane == gm, cand, intmax)
        for k in (1, 2, 4, 8):
            cand = jnp.minimum(cand, _shuffle(cand, lane ^ k))
        idx_buf[...] = cand
        pltpu.sync_copy(idx_buf, cand_hbm.at[wid])

    return sc_kernel


_SUB = 125


@functools.lru_cache(maxsize=None)
def _build_tc_copy(B, S, V):
    W = V // _SUB

    def copy_body(src_ref, dst_ref):
        dst_ref[...] = src_ref[0]

    return pl.pallas_call(
        copy_body,
        grid=(B,),
        in_specs=[pl.BlockSpec((1, 1, _SUB, W), lambda b: (b, S - 1, 0, 0))],
        out_specs=pl.BlockSpec((1, _SUB, W), lambda b: (b, 0, 0)),
        out_shape=jax.ShapeDtypeStruct((B, _SUB, W), jnp.float32),
    )


def kernel(probas, greedy):
    B, S, V = probas.shape
    cand16 = _build_sc_kernel(B, S, V)(probas)
    last_prob = _build_tc_copy(B, S, V)(
        probas.reshape(B, S, _SUB, V // _SUB)).reshape(B, V)
    greedy_candidate = cand16[:, 0]

    def _greedy_branch(ops):
        return ops[0]

    def _sampled_branch(ops):
        key = jax.random.key(42)
        return jax.random.categorical(key, jnp.log(ops[1] + 1e-20), axis=1)

    candidate = lax.cond(jnp.asarray(greedy) != 0, _greedy_branch, _sampled_branch,
                         (greedy_candidate, last_prob))
    return candidate.reshape(B, 1), last_prob


# speedup vs baseline: 4.3532x; 4.3532x over previous
import functools

import jax
import jax.numpy as jnp
from jax import lax
from jax.experimental import pallas as pl
from jax.experimental.pallas import tpu as pltpu
from jax.experimental.pallas import tpu_sc as plsc

_LANES = 16

_GATHER_DNUMS = lax.GatherDimensionNumbers(
    offset_dims=(), collapsed_slice_dims=(0,), start_index_map=(0,))


def _shuffle(x, idx):
    return lax.gather(x, idx[:, None], _GATHER_DNUMS, (1,),
                      indices_are_sorted=False, unique_indices=True,
                      mode=lax.GatherScatterMode.PROMISE_IN_BOUNDS)


@functools.lru_cache(maxsize=None)
def _build_sc_kernel(B, S, V):
    info = plsc.get_sparse_core_info()
    num_cores = info.num_cores
    num_workers = info.num_cores * info.num_subcores
    assert B == num_workers, (B, num_workers)
    assert V % _LANES == 0
    mesh = plsc.VectorSubcoreMesh(core_axis_name="c", subcore_axis_name="s")

    @functools.partial(
        pl.kernel,
        mesh=mesh,
        out_type=(
            jax.ShapeDtypeStruct((B, _LANES), jnp.int32),
            jax.ShapeDtypeStruct((B, V), jnp.float32),
        ),
        scratch_types=(
            pltpu.VMEM((V,), jnp.float32),
            pltpu.VMEM((_LANES,), jnp.int32),
            pltpu.SemaphoreType.DMA,
            pltpu.SemaphoreType.DMA,
        ),
    )
    def sc_kernel(probas_hbm, cand_hbm, prob_hbm, buf, idx_buf, in_sem, out_sem):
        wid = lax.axis_index("s") * num_cores + lax.axis_index("c")

        pltpu.async_copy(probas_hbm.at[wid, S - 1], buf, in_sem).wait()
        out_handle = pltpu.async_copy(buf, prob_hbm.at[wid], out_sem)

        lane = lax.iota(jnp.int32, _LANES)
        intmax = jnp.int32(2**31 - 1)

        K = 5
        group = K * _LANES
        assert V % group == 0

        def body(j, carry):
            ms, mis, iv = carry
            new_ms, new_mis = [], []
            for k in range(K):
                v = buf[pl.ds(j * group + k * _LANES, _LANES)]
                pred = v > ms[k]
                new_ms.append(jnp.where(pred, v, ms[k]))
                new_mis.append(jnp.where(pred, iv, mis[k]))
            return tuple(new_ms), tuple(new_mis), iv + group

        ms0 = tuple(jnp.full((_LANES,), -jnp.inf, jnp.float32) for _ in range(K))
        mis0 = tuple(jnp.zeros((_LANES,), jnp.int32) for _ in range(K))
        ms, mis, _ = lax.fori_loop(0, V // group, body,
                                   (ms0, mis0, jnp.zeros((_LANES,), jnp.int32)),
                                   unroll=8)

        fis = [mis[k] + (k * _LANES) + lane for k in range(K)]

        mlane = ms[0]
        for k in range(1, K):
            mlane = jnp.maximum(mlane, ms[k])
        cand = intmax
        for k in range(K):
            cand = jnp.minimum(cand, jnp.where(ms[k] == mlane, fis[k], intmax))

        gm = mlane
        for k in (1, 2, 4, 8):
            gm = jnp.maximum(gm, _shuffle(gm, lane ^ k))
        cand = jnp.where(mlane == gm, cand, intmax)
        for k in (1, 2, 4, 8):
            cand = jnp.minimum(cand, _shuffle(cand, lane ^ k))
        idx_buf[...] = cand
        pltpu.sync_copy(idx_buf, cand_hbm.at[wid])
        out_handle.wait()

    return sc_kernel


def kernel(probas, greedy):
    B, S, V = probas.shape
    cand16, last_prob = _build_sc_kernel(B, S, V)(probas)
    greedy_candidate = cand16[:, 0]

    def _greedy_branch(ops):
        return ops[0]

    def _sampled_branch(ops):
        key = jax.random.key(42)
        return jax.random.categorical(key, jnp.log(ops[1] + 1e-20), axis=1)

    candidate = lax.cond(jnp.asarray(greedy) != 0, _greedy_branch, _sampled_branch,
                         (greedy_candidate, last_prob))
    return candidate.reshape(B, 1), last_prob


# speedup vs baseline: 5.0020x; 1.1490x over previous
import functools

import jax
import jax.numpy as jnp
from jax import lax
from jax.experimental import pallas as pl
from jax.experimental.pallas import tpu as pltpu
from jax.experimental.pallas import tpu_sc as plsc

_LANES = 16

_GATHER_DNUMS = lax.GatherDimensionNumbers(
    offset_dims=(), collapsed_slice_dims=(0,), start_index_map=(0,))


def _shuffle(x, idx):
    return lax.gather(x, idx[:, None], _GATHER_DNUMS, (1,),
                      indices_are_sorted=False, unique_indices=True,
                      mode=lax.GatherScatterMode.PROMISE_IN_BOUNDS)


@functools.lru_cache(maxsize=None)
def _build_sc_kernel(B, S, V):
    info = plsc.get_sparse_core_info()
    num_cores = info.num_cores
    num_workers = info.num_cores * info.num_subcores
    assert B == num_workers, (B, num_workers)
    assert V % _LANES == 0
    mesh = plsc.VectorSubcoreMesh(core_axis_name="c", subcore_axis_name="s")

    @functools.partial(
        pl.kernel,
        mesh=mesh,
        out_type=(
            jax.ShapeDtypeStruct((B, _LANES), jnp.int32),
            jax.ShapeDtypeStruct((B, V), jnp.float32),
        ),
        scratch_types=(
            pltpu.VMEM((V,), jnp.float32),
            pltpu.VMEM((_LANES,), jnp.int32),
            pltpu.SemaphoreType.DMA,
            pltpu.SemaphoreType.DMA,
        ),
    )
    def sc_kernel(probas_hbm, cand_hbm, prob_hbm, buf, idx_buf, in_sem, out_sem):
        wid = lax.axis_index("s") * num_cores + lax.axis_index("c")

        pltpu.async_copy(probas_hbm.at[wid, S - 1], buf, in_sem).wait()
        out_handle = pltpu.async_copy(buf, prob_hbm.at[wid], out_sem)

        lane = lax.iota(jnp.int32, _LANES)
        intmax = jnp.int32(2**31 - 1)

        K = 10
        group = K * _LANES
        assert V % group == 0

        def body(j, carry):
            ms, mis, iv = carry
            new_ms, new_mis = [], []
            for k in range(K):
                v = buf[pl.ds(j * group + k * _LANES, _LANES)]
                pred = v > ms[k]
                new_ms.append(jnp.where(pred, v, ms[k]))
                new_mis.append(jnp.where(pred, iv, mis[k]))
            return tuple(new_ms), tuple(new_mis), iv + group

        ms0 = tuple(jnp.full((_LANES,), -jnp.inf, jnp.float32) for _ in range(K))
        mis0 = tuple(jnp.zeros((_LANES,), jnp.int32) for _ in range(K))
        ms, mis, _ = lax.fori_loop(0, V // group, body,
                                   (ms0, mis0, jnp.zeros((_LANES,), jnp.int32)),
                                   unroll=4)

        fis = [mis[k] + (k * _LANES) + lane for k in range(K)]

        mlane = ms[0]
        for k in range(1, K):
            mlane = jnp.maximum(mlane, ms[k])
        cand = intmax
        for k in range(K):
            cand = jnp.minimum(cand, jnp.where(ms[k] == mlane, fis[k], intmax))

        gm = mlane
        for k in (1, 2, 4, 8):
            gm = jnp.maximum(gm, _shuffle(gm, lane ^ k))
        cand = jnp.where(mlane == gm, cand, intmax)
        for k in (1, 2, 4, 8):
            cand = jnp.minimum(cand, _shuffle(cand, lane ^ k))
        idx_buf[...] = cand
        pltpu.sync_copy(idx_buf, cand_hbm.at[wid])
        out_handle.wait()

    return sc_kernel


def kernel(probas, greedy):
    B, S, V = probas.shape
    cand16, last_prob = _build_sc_kernel(B, S, V)(probas)
    greedy_candidate = cand16[:, 0]

    def _greedy_branch(ops):
        return ops[0]

    def _sampled_branch(ops):
        key = jax.random.key(42)
        return jax.random.categorical(key, jnp.log(ops[1] + 1e-20), axis=1)

    candidate = lax.cond(jnp.asarray(greedy) != 0, _greedy_branch, _sampled_branch,
                         (greedy_candidate, last_prob))
    return candidate.reshape(B, 1), last_prob
